# baseline (speedup 1.0000x reference)
import functools

import jax
import jax.numpy as jnp
import numpy as np
from jax import lax
from jax.experimental import pallas as pl
from jax.experimental.pallas import tpu as pltpu
from jax.experimental.pallas import tpu_sc as plsc

N_CLS = 1000
MAX_TOK = 67
D = 512
PROMPT_LEN = 5
PREFIX_LEN = 5
HEAD = PREFIX_LEN + PROMPT_LEN
MAX_LEN = 77
NW = 32
BLK = 8
NBLK = N_CLS // BLK
NBLK_PAD = 128
BPW = NBLK_PAD // NW
IDX_PER_BLK = MAX_TOK * BLK
START = MAX_TOK // 2 - PROMPT_LEN // 2
LANES = 16
CHUNKS = ((0, 9), (9, 18), (18, 27), (27, 36), (36, 45), (45, 54),
          (54, 63), (63, 67))
CTX_CHUNK = 3
SIZES = tuple((p1 - p0) * BLK for p0, p1 in CHUNKS) + (HEAD * BLK,)
OFFS = tuple(int(np.cumsum((0,) + SIZES)[k]) for k in range(len(SIZES)))
SCAT_PER_BLK = OFFS[-1] + 0 if False else int(np.sum(SIZES))


def _np_template() -> np.ndarray:
    rows = []
    for p0, p1 in CHUNKS:
        for p in range(p0, p1):
            for j in range(BLK):
                rows.append((HEAD + p) * N_CLS + j)
    for p in range(HEAD):
        for j in range(BLK):
            rows.append(p * N_CLS + j)
    return np.asarray(rows, np.int32)


_TMPL = _np_template()


def _body(idx_hbm, sidx_hbm, table_hbm, prefix_hbm, out_hbm,
          idx_v, sidx_v, gbuf, hbuf, pbuf, scat_refs, g0, g1, o0, o1):
    gsems = (g0, g1)
    osems = (o0, o1)
    wid = lax.axis_index("s") * 2 + lax.axis_index("c")
    pltpu.sync_copy(idx_hbm.at[pl.ds(wid * BPW * IDX_PER_BLK,
                                     BPW * IDX_PER_BLK)], idx_v)
    pltpu.sync_copy(sidx_hbm.at[pl.ds(wid * BPW * SCAT_PER_BLK,
                                      BPW * SCAT_PER_BLK)], sidx_v)
    pltpu.sync_copy(prefix_hbm, pbuf)

    def fill_prefix(p, carry):
        for j16 in range(D // LANES):
            col = pl.ds(j16 * LANES, LANES)
            v = pbuf[p, col]
            for j in range(BLK):
                hbuf[p * BLK + j, col] = v
        return carry

    lax.fori_loop(0, PREFIX_LEN, fill_prefix, 0, unroll=False)

    def block_step(i, carry):
        blk = wid * BPW + i
        sbase = i * SCAT_PER_BLK

        @pl.when(blk < NBLK)
        def _():
            ibase = i * IDX_PER_BLK
            for k in range(len(SIZES)):
                n = SIZES[k]
                offs = sorted(set(list(range(0, n - 15, 16)) + [n - 16]))
                for o in offs:
                    scat_refs[k][pl.ds(o, 16)] = sidx_v[
                        pl.ds(sbase + OFFS[k] + o, 16)]

            def gather(k):
                p0, p1 = CHUNKS[k]
                n = (p1 - p0) * BLK
                return pltpu.async_copy(
                    table_hbm.at[idx_v.at[pl.ds(ibase + p0 * BLK, n)]],
                    gbuf.at[k % 2, pl.ds(0, n)], gsems[k % 2])

            g_pend = {0: gather(0)}
            s_pend = {0: [], 1: []}
            for k, (p0, p1) in enumerate(CHUNKS):
                b = k % 2
                g_pend[k].wait()
                if k + 1 < len(CHUNKS):
                    for cp in s_pend[1 - b]:
                        cp.wait()
                    s_pend[1 - b] = []
                    g_pend[k + 1] = gather(k + 1)
                if k == CTX_CHUNK:
                    def ctx_step(j, carry2):
                        r0 = (START - p0) * BLK + j
                        for j16 in range(D // LANES):
                            col = pl.ds(j16 * LANES, LANES)
                            xs = [gbuf[b, r0 + s * BLK, col]
                                  for s in range(PROMPT_LEN)]
                            mean = (xs[0] + xs[1] + xs[2] + xs[3]
                                    + xs[4]) * 0.2
                            dfs = [x - mean for x in xs]
                            var = (dfs[0] * dfs[0] + dfs[1] * dfs[1]
                                   + dfs[2] * dfs[2] + dfs[3] * dfs[3]
                                   + dfs[4] * dfs[4]) * 0.25
                            yi = jnp.int32(0x5F3759DF) - (
                                lax.bitcast_convert_type(var, jnp.int32) >> 1)
                            y = lax.bitcast_convert_type(yi, jnp.float32)
                            y = y * (1.5 - 0.5 * var * y * y)
                            y = y * (1.5 - 0.5 * var * y * y)
                            y = y * (1.5 - 0.5 * var * y * y)
                            std = var * y
                            scale = 1.0 / (std + 1e-6)
                            for s in range(PROMPT_LEN):
                                hbuf[(PREFIX_LEN + s) * BLK + j, col] = (
                                    dfs[s] * scale)
                        return carry2

                    lax.fori_loop(0, BLK, ctx_step, 0, unroll=False)
                    s_pend[b].append(pltpu.async_copy(
                        hbuf, out_hbm.at[scat_refs[len(CHUNKS)]], osems[b]))
                n = (p1 - p0) * BLK
                s_pend[b].append(pltpu.async_copy(
                    gbuf.at[b, pl.ds(0, n)], out_hbm.at[scat_refs[k]],
                    osems[b]))
            for b in (0, 1):
                for cp in s_pend[b]:
                    cp.wait()

        return carry

    lax.fori_loop(0, BPW, block_step, 0, unroll=False)


def _make_call():
    scat_types = [pltpu.VMEM((n,), jnp.int32) for n in SIZES]

    def wrapper(idx, sidx, table, pref_p):
        def body(idx_hbm, sidx_hbm, table_hbm, prefix_hbm, out_hbm,
                 idx_v, sidx_v, gbuf, hbuf, pbuf, *rest):
            scat_refs = list(rest[:len(SIZES)])
            sems = rest[len(SIZES):]
            return _body(idx_hbm, sidx_hbm, table_hbm, prefix_hbm, out_hbm,
                         idx_v, sidx_v, gbuf, hbuf, pbuf, scat_refs, *sems)

        call = pl.kernel(
            body,
            out_type=jax.ShapeDtypeStruct((MAX_LEN * N_CLS, D), jnp.float32),
            mesh=plsc.VectorSubcoreMesh(core_axis_name="c",
                                        subcore_axis_name="s"),
            scratch_types=[
                pltpu.VMEM((BPW * IDX_PER_BLK,), jnp.int32),
                pltpu.VMEM((BPW * SCAT_PER_BLK,), jnp.int32),
                pltpu.VMEM((2, 9 * BLK, D), jnp.float32),
                pltpu.VMEM((HEAD * BLK, D), jnp.float32),
                pltpu.VMEM((8, D), jnp.float32),
                *scat_types,
                pltpu.SemaphoreType.DMA,
                pltpu.SemaphoreType.DMA,
                pltpu.SemaphoreType.DMA,
                pltpu.SemaphoreType.DMA,
            ],
        )
        return call(idx, sidx, table, pref_p)

    return wrapper


_sc_call = _make_call()


def kernel(token_ids, table, prefix):
    tok_p = jnp.zeros((NBLK_PAD * BLK, MAX_TOK), jnp.int32)
    tok_p = tok_p.at[:N_CLS].set(token_ids.astype(jnp.int32))
    idx = tok_p.reshape(NBLK_PAD, BLK, MAX_TOK).transpose(0, 2, 1).reshape(-1)
    sidx = (jnp.asarray(_TMPL)[None, :]
            + (jnp.arange(NBLK_PAD, dtype=jnp.int32) * BLK)[:, None]
            ).reshape(-1)
    pref_p = jnp.zeros((8, D), jnp.float32).at[:PREFIX_LEN].set(prefix)
    out = _sc_call(idx, sidx, table, pref_p)
    return jnp.transpose(out.reshape(MAX_LEN, N_CLS, D), (1, 0, 2))


# speedup vs baseline: 1.0250x; 1.0250x over previous
import functools

import jax
import jax.numpy as jnp
from jax import lax
from jax.experimental import pallas as pl
from jax.experimental.pallas import tpu as pltpu
from jax.experimental.pallas import tpu_sc as plsc

N_CLS = 1000
MAX_TOK = 67
D = 512
PROMPT_LEN = 5
PREFIX_LEN = 5
HEAD = PREFIX_LEN + PROMPT_LEN
MAX_LEN = 77
NW = 32
BLK = 8
NBLK = N_CLS // BLK
NBLK_PAD = 128
BPW = NBLK_PAD // NW
IDX_PER_BLK = MAX_TOK * BLK
START = MAX_TOK // 2 - PROMPT_LEN // 2
LANES = 16
CHUNKS = ((0, 9), (9, 18), (18, 27), (27, 36), (36, 45), (45, 54),
          (54, 63), (63, 67))
CTX_CHUNK = 3


def _body(idx_hbm, table_hbm, prefix_hbm, out_hbm, idx_v, gbuf, hbuf, pbuf,
          g0, g1, o0, o1):
    gsems = (g0, g1)
    osems = (o0, o1)
    wid = lax.axis_index("s") * 2 + lax.axis_index("c")
    pltpu.sync_copy(idx_hbm.at[pl.ds(wid * BPW * IDX_PER_BLK,
                                     BPW * IDX_PER_BLK)], idx_v)
    pltpu.sync_copy(prefix_hbm, pbuf)

    def fill_prefix(p, carry):
        for j16 in range(D // LANES):
            col = pl.ds(j16 * LANES, LANES)
            v = pbuf[p, col]
            for j in range(BLK):
                hbuf[p * BLK + j, col] = v
        return carry

    lax.fori_loop(0, PREFIX_LEN, fill_prefix, 0, unroll=False)

    def gather_desc(i, k):
        p0, p1 = CHUNKS[k]
        n = (p1 - p0) * BLK
        return pltpu.make_async_copy(
            table_hbm.at[idx_v.at[pl.ds(i * IDX_PER_BLK + p0 * BLK, n)]],
            gbuf.at[k % 2, pl.ds(0, n)], gsems[k % 2])

    gather_desc(0, 0).start()

    def block_step(i, carry):
        blk = wid * BPW + i
        c0 = blk * BLK

        @pl.when(blk < NBLK)
        def _():
            s_pend = {0: [], 1: []}
            g_pend = {}
            for k, (p0, p1) in enumerate(CHUNKS):
                b = k % 2
                (g_pend[k] if k in g_pend else gather_desc(i, 0)).wait()
                if k + 1 < len(CHUNKS):
                    for cp in s_pend[1 - b]:
                        cp.wait()
                    s_pend[1 - b] = []
                    g_pend[k + 1] = gather_desc(i, k + 1)
                    g_pend[k + 1].start()
                if k == CTX_CHUNK:
                    def ctx_step(j, carry2):
                        r0 = (START - p0) * BLK + j
                        for j16 in range(D // LANES):
                            col = pl.ds(j16 * LANES, LANES)
                            xs = [gbuf[b, r0 + s * BLK, col]
                                  for s in range(PROMPT_LEN)]
                            mean = (xs[0] + xs[1] + xs[2] + xs[3]
                                    + xs[4]) * 0.2
                            dfs = [x - mean for x in xs]
                            var = (dfs[0] * dfs[0] + dfs[1] * dfs[1]
                                   + dfs[2] * dfs[2] + dfs[3] * dfs[3]
                                   + dfs[4] * dfs[4]) * 0.25
                            yi = jnp.int32(0x5F3759DF) - (
                                lax.bitcast_convert_type(var, jnp.int32) >> 1)
                            y = lax.bitcast_convert_type(yi, jnp.float32)
                            y = y * (1.5 - 0.5 * var * y * y)
                            y = y * (1.5 - 0.5 * var * y * y)
                            y = y * (1.5 - 0.5 * var * y * y)
                            std = var * y
                            scale = 1.0 / (std + 1e-6)
                            for s in range(PROMPT_LEN):
                                hbuf[(PREFIX_LEN + s) * BLK + j, col] = (
                                    dfs[s] * scale)
                        return carry2

                    lax.fori_loop(0, BLK, ctx_step, 0, unroll=False)
                    for p in range(HEAD):
                        s_pend[b].append(pltpu.async_copy(
                            hbuf.at[pl.ds(p * BLK, BLK)],
                            out_hbm.at[p, pl.ds(c0, BLK)], osems[b]))
                for p in range(p0, p1):
                    s_pend[b].append(pltpu.async_copy(
                        gbuf.at[b, pl.ds((p - p0) * BLK, BLK)],
                        out_hbm.at[HEAD + p, pl.ds(c0, BLK)], osems[b]))

            for cp in s_pend[0]:
                cp.wait()

            @pl.when((i + 1 < BPW) & (wid * BPW + i + 1 < NBLK))
            def _prefetch():
                gather_desc(i + 1, 0).start()

            for cp in s_pend[1]:
                cp.wait()

        return carry

    lax.fori_loop(0, BPW, block_step, 0, unroll=False)


_sc_call = functools.partial(
    pl.kernel,
    out_type=jax.ShapeDtypeStruct((MAX_LEN, N_CLS, D), jnp.float32),
    mesh=plsc.VectorSubcoreMesh(core_axis_name="c", subcore_axis_name="s"),
    scratch_types=[
        pltpu.VMEM((BPW * IDX_PER_BLK,), jnp.int32),
        pltpu.VMEM((2, 9 * BLK, D), jnp.float32),
        pltpu.VMEM((HEAD * BLK, D), jnp.float32),
        pltpu.VMEM((8, D), jnp.float32),
        pltpu.SemaphoreType.DMA,
        pltpu.SemaphoreType.DMA,
        pltpu.SemaphoreType.DMA,
        pltpu.SemaphoreType.DMA,
    ],
)(_body)


def kernel(token_ids, table, prefix):
    tok_p = jnp.zeros((NBLK_PAD * BLK, MAX_TOK), jnp.int32)
    tok_p = tok_p.at[:N_CLS].set(token_ids.astype(jnp.int32))
    idx = tok_p.reshape(NBLK_PAD, BLK, MAX_TOK).transpose(0, 2, 1).reshape(-1)
    pref_p = jnp.zeros((8, D), jnp.float32).at[:PREFIX_LEN].set(prefix)
    out = _sc_call(idx, table, pref_p)
    return jnp.transpose(out, (1, 0, 2))


# speedup vs baseline: 1.0562x; 1.0305x over previous
import functools

import jax
import jax.numpy as jnp
from jax import lax
from jax.experimental import pallas as pl
from jax.experimental.pallas import tpu as pltpu
from jax.experimental.pallas import tpu_sc as plsc

N_CLS = 1000
MAX_TOK = 67
D = 512
PROMPT_LEN = 5
PREFIX_LEN = 5
HEAD = PREFIX_LEN + PROMPT_LEN
MAX_LEN = 77
NW = 32
BLK = 8
NBLK = N_CLS // BLK
NBLK_PAD = 128
BPW = NBLK_PAD // NW
IDX_PER_BLK = MAX_TOK * BLK
START = MAX_TOK // 2 - PROMPT_LEN // 2
LANES = 16
CHUNKS = ((0, 9), (9, 18), (18, 27), (27, 36), (36, 45), (45, 54),
          (54, 63), (63, 67))
CTX_CHUNK = 3


def _body(idx_hbm, table_hbm, prefix_hbm, out_hbm, idx_v, gbuf, hbuf, pbuf,
          g0, g1, o0, o1):
    gsems = (g0, g1)
    osems = (o0, o1)
    wid = lax.axis_index("s") * 2 + lax.axis_index("c")
    pltpu.sync_copy(idx_hbm.at[pl.ds(wid * BPW * IDX_PER_BLK,
                                     BPW * IDX_PER_BLK)], idx_v)
    pltpu.sync_copy(prefix_hbm, pbuf)

    def fill_prefix(p, carry):
        for j16 in range(D // LANES):
            col = pl.ds(j16 * LANES, LANES)
            v = pbuf[p, col]
            for j in range(BLK):
                hbuf[p * BLK + j, col] = v
        return carry

    lax.fori_loop(0, PREFIX_LEN, fill_prefix, 0, unroll=False)

    def gather_desc(i, k):
        p0, p1 = CHUNKS[k]
        n = (p1 - p0) * BLK
        return pltpu.make_async_copy(
            table_hbm.at[idx_v.at[pl.ds(i * IDX_PER_BLK + p0 * BLK, n)]],
            gbuf.at[k % 2, pl.ds(0, n)], gsems[k % 2])

    gather_desc(0, 0).start()

    def block_step(i, carry):
        blk = wid * BPW + i
        c0 = blk * BLK

        @pl.when(blk < NBLK)
        def _():
            s_pend = {0: [], 1: []}
            g_pend = {}
            for k, (p0, p1) in enumerate(CHUNKS):
                b = k % 2
                (g_pend[k] if k in g_pend else gather_desc(i, 0)).wait()
                if k + 1 < len(CHUNKS):
                    for cp in s_pend[1 - b]:
                        cp.wait()
                    s_pend[1 - b] = []
                    g_pend[k + 1] = gather_desc(i, k + 1)
                    g_pend[k + 1].start()
                for p in range(p0, p1):
                    s_pend[b].append(pltpu.async_copy(
                        gbuf.at[b, pl.ds((p - p0) * BLK, BLK)],
                        out_hbm.at[HEAD + p, pl.ds(c0, BLK)], osems[b]))
                if k == CTX_CHUNK:
                    def ctx_step(j, carry2):
                        r0 = (START - p0) * BLK + j
                        for j16 in range(D // LANES):
                            col = pl.ds(j16 * LANES, LANES)
                            xs = [gbuf[b, r0 + s * BLK, col]
                                  for s in range(PROMPT_LEN)]
                            mean = (xs[0] + xs[1] + xs[2] + xs[3]
                                    + xs[4]) * 0.2
                            dfs = [x - mean for x in xs]
                            var = (dfs[0] * dfs[0] + dfs[1] * dfs[1]
                                   + dfs[2] * dfs[2] + dfs[3] * dfs[3]
                                   + dfs[4] * dfs[4]) * 0.25
                            yi = jnp.int32(0x5F3759DF) - (
                                lax.bitcast_convert_type(var, jnp.int32) >> 1)
                            y = lax.bitcast_convert_type(yi, jnp.float32)
                            y = y * (1.5 - 0.5 * var * y * y)
                            y = y * (1.5 - 0.5 * var * y * y)
                            std = var * y
                            scale = 1.0 / (std + 1e-6)
                            for s in range(PROMPT_LEN):
                                hbuf[(PREFIX_LEN + s) * BLK + j, col] = (
                                    dfs[s] * scale)
                        return carry2

                    lax.fori_loop(0, BLK, ctx_step, 0, unroll=False)
                    for p in range(HEAD):
                        s_pend[b].append(pltpu.async_copy(
                            hbuf.at[pl.ds(p * BLK, BLK)],
                            out_hbm.at[p, pl.ds(c0, BLK)], osems[b]))

            for cp in s_pend[0]:
                cp.wait()

            @pl.when((i + 1 < BPW) & (wid * BPW + i + 1 < NBLK))
            def _prefetch():
                gather_desc(i + 1, 0).start()

            for cp in s_pend[1]:
                cp.wait()

        return carry

    lax.fori_loop(0, BPW, block_step, 0, unroll=False)


_sc_call = functools.partial(
    pl.kernel,
    out_type=jax.ShapeDtypeStruct((MAX_LEN, N_CLS, D), jnp.float32),
    mesh=plsc.VectorSubcoreMesh(core_axis_name="c", subcore_axis_name="s"),
    scratch_types=[
        pltpu.VMEM((BPW * IDX_PER_BLK,), jnp.int32),
        pltpu.VMEM((2, 9 * BLK, D), jnp.float32),
        pltpu.VMEM((HEAD * BLK, D), jnp.float32),
        pltpu.VMEM((8, D), jnp.float32),
        pltpu.SemaphoreType.DMA,
        pltpu.SemaphoreType.DMA,
        pltpu.SemaphoreType.DMA,
        pltpu.SemaphoreType.DMA,
    ],
)(_body)


def kernel(token_ids, table, prefix):
    tok_p = jnp.zeros((NBLK_PAD * BLK, MAX_TOK), jnp.int32)
    tok_p = tok_p.at[:N_CLS].set(token_ids.astype(jnp.int32))
    idx = tok_p.reshape(NBLK_PAD, BLK, MAX_TOK).transpose(0, 2, 1).reshape(-1)
    pref_p = jnp.zeros((8, D), jnp.float32).at[:PREFIX_LEN].set(prefix)
    out = _sc_call(idx, table, pref_p)
    return jnp.transpose(out, (1, 0, 2))
